# baseline (speedup 1.0000x reference)
import functools

import jax
import jax.numpy as jnp
from jax import lax
from jax.experimental import pallas as pl
from jax.experimental.pallas import tpu as pltpu
from jax.experimental.pallas import tpu_sc as plsc

EPS = 1e-5
TM = 512

_SC_CORES = 2
_SC_SUBCORES = 16
_SC_WORKERS = _SC_CORES * _SC_SUBCORES


def _dist_argmin_body(x_ref, mean_ref, denom_ref, gamma_ref, beta_ref,
                      cb_ref, b2_ref, a2_ref, idx_ref):
    xn = (x_ref[...] - mean_ref[...]) / denom_ref[...] * gamma_ref[...] + beta_ref[...]
    s = lax.dot_general(xn, cb_ref[...], (((1,), (1,)), ((), ())),
                        preferred_element_type=jnp.float32)
    d2 = a2_ref[...] + b2_ref[...] - 2.0 * s
    dist = jnp.sqrt(jnp.maximum(d2, 0.0))
    m = jnp.min(dist, axis=1, keepdims=True)
    iota = lax.broadcasted_iota(jnp.int32, dist.shape, 1)
    idx_ref[...] = jnp.min(jnp.where(dist == m, iota, dist.shape[1]), axis=1)


def _nearest_indices(xn_inputs, n_tokens, dim, num_codes):
    x, mean, denom, gamma2, beta2, codebook, b2, a2 = xn_inputs
    return pl.pallas_call(
        _dist_argmin_body,
        grid=(n_tokens // TM,),
        in_specs=[
            pl.BlockSpec((TM, dim), lambda i: (i, 0)),
            pl.BlockSpec((1, dim), lambda i: (0, 0)),
            pl.BlockSpec((1, dim), lambda i: (0, 0)),
            pl.BlockSpec((1, dim), lambda i: (0, 0)),
            pl.BlockSpec((1, dim), lambda i: (0, 0)),
            pl.BlockSpec((num_codes, dim), lambda i: (0, 0)),
            pl.BlockSpec((1, num_codes), lambda i: (0, 0)),
            pl.BlockSpec((TM, 1), lambda i: (i, 0)),
        ],
        out_specs=pl.BlockSpec((TM,), lambda i: (i,)),
        out_shape=jax.ShapeDtypeStruct((n_tokens,), jnp.int32),
    )(x, mean, denom, gamma2, beta2, codebook, b2, a2)


@functools.lru_cache(maxsize=None)
def _make_sc_gather(num_codes, dim, n_tokens):
    b_per_w = n_tokens // _SC_WORKERS
    mesh = plsc.VectorSubcoreMesh(core_axis_name="c", subcore_axis_name="s")

    @functools.partial(
        pl.kernel, mesh=mesh,
        out_type=jax.ShapeDtypeStruct((n_tokens, dim), jnp.float32),
        scratch_types=[
            pltpu.VMEM((b_per_w,), jnp.int32),
            pltpu.VMEM((b_per_w, dim), jnp.float32),
            pltpu.SemaphoreType.DMA,
        ],
    )
    def gather(table_hbm, idx_hbm, out_hbm, idx_v, rows_v, sem):
        wid = lax.axis_index("s") * _SC_CORES + lax.axis_index("c")
        base = wid * b_per_w
        pltpu.sync_copy(idx_hbm.at[pl.ds(base, b_per_w)], idx_v)
        pltpu.async_copy(table_hbm.at[idx_v], rows_v, sem).wait()
        pltpu.sync_copy(rows_v, out_hbm.at[pl.ds(base, b_per_w)])

    return gather


def kernel(x, bn_gamma, bn_beta, codebook):
    n_tokens, dim = x.shape
    num_codes = codebook.shape[0]
    mean = jnp.mean(x, axis=0, keepdims=True)
    var = jnp.mean((x - mean) ** 2, axis=0, keepdims=True)
    denom = jnp.sqrt(var + EPS)
    b2 = jnp.sum(codebook * codebook, axis=-1)[None, :]
    xn_stat = (x - mean) / denom * bn_gamma + bn_beta
    a2 = jnp.sum(xn_stat * xn_stat, axis=-1, keepdims=True)
    indices = _nearest_indices(
        (x, mean, denom, bn_gamma[None, :], bn_beta[None, :], codebook, b2, a2),
        n_tokens, dim, num_codes)
    codes = _make_sc_gather(num_codes, dim, n_tokens)(codebook, indices)
    return codes, indices.reshape(n_tokens, 1)
